# baseline (speedup 1.0000x reference)
import jax
import jax.numpy as jnp
from jax import lax
from jax.experimental import pallas as pl
from jax.experimental.pallas import tpu as pltpu
from jax.experimental.pallas import tpu_sc as plsc

N = 10000
E = 160000
K = 16
NUM_LAYERS = 4
F = 128
NC, NS = 2, 16
NW = NC * NS
CH = 128
NCHUNK = E // CH
JPW = -(-NCHUNK // NW)
NCHUNKP = JPW * NW
EP = NCHUNKP * CH
NPAD = 10240
RPT = NPAD // NS
ZR = 40
NBUF = 3
NBUF_S = 2

_MESH = plsc.VectorSubcoreMesh(core_axis_name="c", subcore_axis_name="s",
                               num_cores=NC, num_subcores=NS)


def _make_gather(d):
  def body(p_hbm, q_hbm, srcp_hbm, dstp_hbm, gs_hbm, gd_hbm, idx_s, idx_d,
           *bufs):
    rows = bufs[:NBUF]
    gsem = bufs[NBUF:2 * NBUF]
    wsem = bufs[2 * NBUF:3 * NBUF]
    wid = lax.axis_index("s") * NC + lax.axis_index("c")
    ebase = pl.multiple_of(wid * (JPW * CH), 8)
    pltpu.sync_copy(srcp_hbm.at[pl.ds(ebase, JPW * CH)], idx_s)
    pltpu.sync_copy(dstp_hbm.at[pl.ds(ebase, JPW * CH)], idx_d)

    NJ = 2 * JPW
    gdesc = [None] * NJ
    wdesc = [None] * NJ

    def start(j):
      s = j % NBUF
      jj = j if j < JPW else j - JPW
      tbl = p_hbm if j < JPW else q_hbm
      ia = idx_s if j < JPW else idx_d
      gdesc[j] = pltpu.async_copy(
          tbl.at[ia.at[pl.ds(jj * CH, CH)]], rows[s], gsem[s])

    def writeback(j):
      s = j % NBUF
      jj = j if j < JPW else j - JPW
      out = gs_hbm if j < JPW else gd_hbm
      off = pl.multiple_of(wid * (JPW * CH) + jj * CH, 8)
      wdesc[j] = pltpu.async_copy(rows[s], out.at[pl.ds(off, CH)], wsem[s])

    for j in range(NJ):
      if j >= NBUF:
        wdesc[j - NBUF].wait()
      start(j)
      if j >= 1:
        gdesc[j - 1].wait()
        writeback(j - 1)
    gdesc[NJ - 1].wait()
    writeback(NJ - 1)
    for j in range(NJ - NBUF, NJ):
      wdesc[j].wait()

  return pl.kernel(
      body,
      out_type=[jax.ShapeDtypeStruct((EP, d), jnp.float32)] * 2,
      mesh=_MESH,
      compiler_params=pltpu.CompilerParams(use_tc_tiling_on_sc=(d == F)),
      scratch_types=(
          [pltpu.VMEM((JPW * CH,), jnp.int32)] * 2
          + [pltpu.VMEM((CH, d), jnp.float32)] * NBUF
          + [pltpu.SemaphoreType.DMA] * (2 * NBUF)
      ),
  )


_gather_f = _make_gather(F)
_gather_g = _make_gather(16)


def _make_scatter(d):
  def body(v_hbm, dst2p_hbm, out_hbm, idx_v, zbuf, *bufs):
    rows = bufs[:NBUF_S]
    lsem = bufs[NBUF_S:2 * NBUF_S]
    ssem = bufs[2 * NBUF_S:3 * NBUF_S]
    s_sh = bufs[3 * NBUF_S]
    cid = lax.axis_index("c")
    sid = lax.axis_index("s")
    wid = sid * NC + cid

    for r in range(ZR):
      for col in range(d // 16):
        zbuf[r, pl.ds(col * 16, 16)] = jnp.zeros((16,), jnp.float32)
    for i in range(RPT // ZR):
      pltpu.sync_copy(zbuf, s_sh.at[pl.ds(sid * RPT + i * ZR, ZR)])

    c0 = wid * JPW
    pltpu.sync_copy(dst2p_hbm.at[pl.ds(c0, JPW)], idx_v)
    plsc.subcore_barrier()

    ldesc = [None] * JPW
    sdesc = [None] * JPW

    def load(j):
      s = j % NBUF_S
      off = pl.multiple_of((wid * JPW + j) * CH, 8)
      ldesc[j] = pltpu.async_copy(v_hbm.at[pl.ds(off, CH)], rows[s], lsem[s])

    def scat(j):
      s = j % NBUF_S
      sdesc[j] = pltpu.async_copy(rows[s], s_sh.at[idx_v.at[j]], ssem[s],
                                  add=True)

    for j in range(JPW):
      if j >= NBUF_S:
        sdesc[j - NBUF_S].wait()
      load(j)
      if j >= 1:
        ldesc[j - 1].wait()
        scat(j - 1)
    ldesc[JPW - 1].wait()
    scat(JPW - 1)
    for j in range(JPW - NBUF_S, JPW):
      sdesc[j].wait()

    plsc.subcore_barrier()
    pltpu.sync_copy(
        s_sh.at[pl.ds(sid * RPT, RPT)],
        out_hbm.at[pl.ds(cid * NPAD + sid * RPT, RPT)],
    )

  return pl.kernel(
      body,
      out_type=jax.ShapeDtypeStruct((NC * NPAD, d), jnp.float32),
      mesh=_MESH,
      compiler_params=pltpu.CompilerParams(use_tc_tiling_on_sc=(d == F)),
      scratch_types=(
          [pltpu.VMEM((JPW, CH), jnp.int32),
           pltpu.VMEM((ZR, d), jnp.float32)]
          + [pltpu.VMEM((CH, d), jnp.float32)] * NBUF_S
          + [pltpu.SemaphoreType.DMA] * (2 * NBUF_S)
          + [pltpu.MemorySpace.VMEM_SHARED((NPAD, d), jnp.float32)]
      ),
  )


_scatter_f = _make_scatter(F)
_scatter_g = _make_scatter(16)


_BE = 2000


def _geo_body(ts_ref, td_ref, crel_ref, cm_ref, dist_ref, em_ref, em16_ref):
  ts = ts_ref[...]
  td = td_ref[...]
  diff = ts - td
  d2 = jnp.sum(diff * diff * crel_ref[...], axis=1, keepdims=True)
  dist_ref[...] = jnp.sqrt(d2 + 1e-8)
  am = jnp.sum(ts * cm_ref[...], axis=1, keepdims=True)
  bm = jnp.sum(td * cm_ref[...], axis=1, keepdims=True)
  em = am * bm
  em_ref[...] = em
  em16_ref[...] = jnp.broadcast_to(em, em16_ref.shape)


_geo_call = pl.pallas_call(
    _geo_body,
    grid=(E // _BE,),
    in_specs=[
        pl.BlockSpec((_BE, 16), lambda i: (i, 0)),
        pl.BlockSpec((_BE, 16), lambda i: (i, 0)),
        pl.BlockSpec((1, 16), lambda i: (0, 0)),
        pl.BlockSpec((1, 16), lambda i: (0, 0)),
    ],
    out_specs=[
        pl.BlockSpec((_BE, 1), lambda i: (i, 0)),
        pl.BlockSpec((_BE, 1), lambda i: (i, 0)),
        pl.BlockSpec((_BE, 16), lambda i: (i, 0)),
    ],
    out_shape=[
        jax.ShapeDtypeStruct((E, 1), jnp.float32),
        jax.ShapeDtypeStruct((E, 1), jnp.float32),
        jax.ShapeDtypeStruct((EP, 16), jnp.float32),
    ],
)


_BN = 2000


def _pq_body(node_ref, lat_ref, r_ref, a1a, a1b, a2a, a2b, ag3, b1, p_ref, q_ref):
  node = node_ref[...]
  latv = lat_ref[...]
  rg = jnp.dot(r_ref[...], ag3[...], preferred_element_type=jnp.float32)
  p = (jnp.dot(node, a1a[...], preferred_element_type=jnp.float32)
       + jnp.dot(latv, a1b[...], preferred_element_type=jnp.float32))
  q = (jnp.dot(node, a2a[...], preferred_element_type=jnp.float32)
       + jnp.dot(latv, a2b[...], preferred_element_type=jnp.float32))
  p_ref[...] = p + rg + b1[...]
  q_ref[...] = q - rg


_pq_call = pl.pallas_call(
    _pq_body,
    grid=(N // _BN,),
    in_specs=[
        pl.BlockSpec((_BN, F), lambda i: (i, 0)),
        pl.BlockSpec((_BN, F), lambda i: (i, 0)),
        pl.BlockSpec((_BN, 3), lambda i: (i, 0)),
        pl.BlockSpec((F, F), lambda i: (0, 0)),
        pl.BlockSpec((F, F), lambda i: (0, 0)),
        pl.BlockSpec((F, F), lambda i: (0, 0)),
        pl.BlockSpec((F, F), lambda i: (0, 0)),
        pl.BlockSpec((3, F), lambda i: (0, 0)),
        pl.BlockSpec((1, F), lambda i: (0, 0)),
    ],
    out_specs=[
        pl.BlockSpec((_BN, F), lambda i: (i, 0)),
        pl.BlockSpec((_BN, F), lambda i: (i, 0)),
    ],
    out_shape=[jax.ShapeDtypeStruct((N, F), jnp.float32)] * 2,
)


def _edge_body(gs_ref, gd_ref, z_ref, dist_ref, em_ref, az, we, ag4, be,
               zo_ref, hm_ref):
  z = z_ref[...]
  pre = (gs_ref[...] + gd_ref[...]
         + jnp.dot(z, az[...], preferred_element_type=jnp.float32)
         + dist_ref[...] * ag4[...])
  hmid = jnp.maximum(pre, 0.0)
  em = em_ref[...]
  hm_ref[...] = hmid * em
  zo_ref[...] = z + (jnp.dot(hmid, we[...], preferred_element_type=jnp.float32)
                     + be[...]) * em


def _edge_body_last(gs_ref, gd_ref, z_ref, dist_ref, em_ref, az, ag4, hm_ref):
  pre = (gs_ref[...] + gd_ref[...]
         + jnp.dot(z_ref[...], az[...], preferred_element_type=jnp.float32)
         + dist_ref[...] * ag4[...])
  hm_ref[...] = jnp.maximum(pre, 0.0) * em_ref[...]


_EDGE_IN_SPECS = [
    pl.BlockSpec((_BE, F), lambda i: (i, 0)),
    pl.BlockSpec((_BE, F), lambda i: (i, 0)),
    pl.BlockSpec((_BE, F), lambda i: (i, 0)),
    pl.BlockSpec((_BE, 1), lambda i: (i, 0)),
    pl.BlockSpec((_BE, 1), lambda i: (i, 0)),
    pl.BlockSpec((F, F), lambda i: (0, 0)),
]

_edge_call = pl.pallas_call(
    _edge_body,
    grid=(E // _BE,),
    in_specs=_EDGE_IN_SPECS + [
        pl.BlockSpec((F, F), lambda i: (0, 0)),
        pl.BlockSpec((1, F), lambda i: (0, 0)),
        pl.BlockSpec((1, F), lambda i: (0, 0)),
    ],
    out_specs=[
        pl.BlockSpec((_BE, F), lambda i: (i, 0)),
        pl.BlockSpec((_BE, F), lambda i: (i, 0)),
    ],
    out_shape=[
        jax.ShapeDtypeStruct((E, F), jnp.float32),
        jax.ShapeDtypeStruct((EP, F), jnp.float32),
    ],
)

_edge_call_last = pl.pallas_call(
    _edge_body_last,
    grid=(E // _BE,),
    in_specs=_EDGE_IN_SPECS + [pl.BlockSpec((1, F), lambda i: (0, 0))],
    out_specs=pl.BlockSpec((_BE, F), lambda i: (i, 0)),
    out_shape=jax.ShapeDtypeStruct((EP, F), jnp.float32),
)


def _joint_parts(node, latv, s0, s1, d0, d1, mask, w2, b2k):
  s = s0 + s1
  deg = jnp.sum(d0 + d1, axis=1, keepdims=True) * (1.0 / 16.0)
  agg = (jnp.dot(s, w2, preferred_element_type=jnp.float32) * (1.0 / K)
         + deg * b2k)
  jn = (node + agg[:, 0:F]) * mask
  jl = (latv + agg[:, F:2 * F]) * mask
  jz = agg[:, 2 * F:3 * F] * mask
  return jnp.concatenate([jn, jl, jz], axis=1)


def _node_body(node_ref, lat_ref, s0_ref, s1_ref, d0_ref, d1_ref, mask_ref,
               w2, b2k, wlat, wnode, gamma, beta, lo_ref, no_ref):
  node = node_ref[...]
  latv = lat_ref[...]
  joint = _joint_parts(node, latv, s0_ref[...], s1_ref[...], d0_ref[...],
                       d1_ref[...], mask_ref[...], w2[...], b2k[...])
  lo_ref[...] = latv + jnp.dot(joint, wlat[...],
                               preferred_element_type=jnp.float32)
  npre = node + jnp.dot(joint, wnode[...], preferred_element_type=jnp.float32)
  mu = jnp.mean(npre, axis=1, keepdims=True)
  var = jnp.mean((npre - mu) ** 2, axis=1, keepdims=True)
  no_ref[...] = (npre - mu) / jnp.sqrt(var + 1e-5) * gamma[...] + beta[...]


def _node_body_last(node_ref, lat_ref, s0_ref, s1_ref, d0_ref, d1_ref,
                    mask_ref, w2, b2k, wlat, lo_ref):
  latv = lat_ref[...]
  joint = _joint_parts(node_ref[...], latv, s0_ref[...], s1_ref[...],
                       d0_ref[...], d1_ref[...], mask_ref[...], w2[...],
                       b2k[...])
  lo_ref[...] = latv + jnp.dot(joint, wlat[...],
                               preferred_element_type=jnp.float32)


_NODE_IN_SPECS = [
    pl.BlockSpec((_BN, F), lambda i: (i, 0)),
    pl.BlockSpec((_BN, F), lambda i: (i, 0)),
    pl.BlockSpec((_BN, F), lambda i: (i, 0)),
    pl.BlockSpec((_BN, F), lambda i: (i, 0)),
    pl.BlockSpec((_BN, 16), lambda i: (i, 0)),
    pl.BlockSpec((_BN, 16), lambda i: (i, 0)),
    pl.BlockSpec((_BN, 1), lambda i: (i, 0)),
    pl.BlockSpec((F, 3 * F), lambda i: (0, 0)),
    pl.BlockSpec((1, 3 * F), lambda i: (0, 0)),
    pl.BlockSpec((3 * F, F), lambda i: (0, 0)),
]

_node_call = pl.pallas_call(
    _node_body,
    grid=(N // _BN,),
    in_specs=_NODE_IN_SPECS + [
        pl.BlockSpec((3 * F, F), lambda i: (0, 0)),
        pl.BlockSpec((1, F), lambda i: (0, 0)),
        pl.BlockSpec((1, F), lambda i: (0, 0)),
    ],
    out_specs=[
        pl.BlockSpec((_BN, F), lambda i: (i, 0)),
        pl.BlockSpec((_BN, F), lambda i: (i, 0)),
    ],
    out_shape=[jax.ShapeDtypeStruct((N, F), jnp.float32)] * 2,
)

_node_call_last = pl.pallas_call(
    _node_body_last,
    grid=(N // _BN,),
    in_specs=_NODE_IN_SPECS,
    out_specs=pl.BlockSpec((_BN, F), lambda i: (i, 0)),
    out_shape=jax.ShapeDtypeStruct((N, F), jnp.float32),
)


def kernel(latent_features, node_features, edge_features, rigids_t, node_mask,
           params, edge_index):
  src = edge_index[1]
  dst = edge_index[0]
  srcp = jnp.pad(src, (0, EP - E))
  dstp = jnp.pad(dst, (0, EP - E))
  dst2p = jnp.pad(dst, (0, EP - E), constant_values=N).reshape(NCHUNKP, CH)

  tm = jnp.concatenate(
      [rigids_t, node_mask[:, None], jnp.zeros((N, 12), jnp.float32)], axis=1)
  ts, td = _gather_g(tm, tm, srcp, dstp)
  crel = jnp.concatenate(
      [jnp.ones((1, 3), jnp.float32), jnp.zeros((1, 13), jnp.float32)], axis=1)
  cm = jnp.concatenate(
      [jnp.zeros((1, 3), jnp.float32), jnp.ones((1, 1), jnp.float32),
       jnp.zeros((1, 12), jnp.float32)], axis=1)
  dist, em, em16 = _geo_call(ts, td, crel, cm)
  deg16 = _scatter_g(em16, dst2p)

  node = node_features
  lat = latent_features
  z = edge_features
  maskc = node_mask[:, None]

  for l in range(NUM_LAYERS):
    W1 = params['W1'][l]
    a1a, a1b = W1[0:128], W1[128:256]
    a2a, a2b = W1[384:512], W1[512:640]
    az, ag3, ag4 = W1[768:896], W1[896:899], W1[899:900]
    b1 = params['b1'][l][None]
    p, q = _pq_call(node, lat, rigids_t, a1a, a1b, a2a, a2b, ag3, b1)
    gs, gd = _gather_f(p, q, srcp, dstp)
    last = l == NUM_LAYERS - 1
    if last:
      hm = _edge_call_last(gs, gd, z, dist, em, az, ag4)
    else:
      z, hm = _edge_call(gs, gd, z, dist, em, az, params['We'][l],
                         ag4, params['be'][l][None])
    s = _scatter_f(hm, dst2p)
    b2k = (params['b2'][l] / K)[None]
    node_args = (node, lat, s[0:N], s[NPAD:NPAD + N],
                 deg16[0:N], deg16[NPAD:NPAD + N], maskc,
                 params['W2'][l], b2k, params['Wlat'][l])
    if last:
      lat = _node_call_last(*node_args)
    else:
      lat, node = _node_call(*node_args, params['Wnode'][l],
                             params['gamma'][l][None], params['beta'][l][None])
  return lat


# speedup vs baseline: 1.7667x; 1.7667x over previous
import jax
import jax.numpy as jnp
from jax import lax
from jax.experimental import pallas as pl
from jax.experimental.pallas import tpu as pltpu
from jax.experimental.pallas import tpu_sc as plsc

N = 10000
E = 160000
K = 16
NUM_LAYERS = 4
F = 128
NC, NS = 2, 16
NW = NC * NS
CH = 128
NCHUNK = E // CH
JPW = -(-NCHUNK // NW)
NCHUNKP = JPW * NW
EP = NCHUNKP * CH
NPAD = 10240
RPT = NPAD // NS
ZR = 40
NBUF = 3
NBUF_S = 2

_MESH = plsc.VectorSubcoreMesh(core_axis_name="c", subcore_axis_name="s",
                               num_cores=NC, num_subcores=NS)


def _make_gather(d):
  def body(p_hbm, q_hbm, srcp_hbm, dstp_hbm, gs_hbm, gd_hbm, idx_s, idx_d,
           *bufs):
    rows = bufs[:NBUF]
    gsem = bufs[NBUF:2 * NBUF]
    wsem = bufs[2 * NBUF:3 * NBUF]
    wid = lax.axis_index("s") * NC + lax.axis_index("c")
    ebase = pl.multiple_of(wid * (JPW * CH), 8)
    pltpu.sync_copy(srcp_hbm.at[pl.ds(ebase, JPW * CH)], idx_s)
    pltpu.sync_copy(dstp_hbm.at[pl.ds(ebase, JPW * CH)], idx_d)

    NJ = 2 * JPW
    gdesc = [None] * NJ
    wdesc = [None] * NJ

    def start(j):
      s = j % NBUF
      jj = j if j < JPW else j - JPW
      tbl = p_hbm if j < JPW else q_hbm
      ia = idx_s if j < JPW else idx_d
      gdesc[j] = pltpu.async_copy(
          tbl.at[ia.at[pl.ds(jj * CH, CH)]], rows[s], gsem[s])

    def writeback(j):
      s = j % NBUF
      jj = j if j < JPW else j - JPW
      out = gs_hbm if j < JPW else gd_hbm
      off = pl.multiple_of(wid * (JPW * CH) + jj * CH, 8)
      wdesc[j] = pltpu.async_copy(rows[s], out.at[pl.ds(off, CH)], wsem[s])

    for j in range(NJ):
      if j >= NBUF:
        wdesc[j - NBUF].wait()
      start(j)
      if j >= 1:
        gdesc[j - 1].wait()
        writeback(j - 1)
    gdesc[NJ - 1].wait()
    writeback(NJ - 1)
    for j in range(NJ - NBUF, NJ):
      wdesc[j].wait()

  return pl.kernel(
      body,
      out_type=[jax.ShapeDtypeStruct((EP, d), jnp.float32)] * 2,
      mesh=_MESH,
      compiler_params=pltpu.CompilerParams(use_tc_tiling_on_sc=(d == F)),
      scratch_types=(
          [pltpu.VMEM((JPW * CH,), jnp.int32)] * 2
          + [pltpu.VMEM((CH, d), jnp.float32)] * NBUF
          + [pltpu.SemaphoreType.DMA] * (2 * NBUF)
      ),
  )


_gather_f = _make_gather(F)
_gather_g = _make_gather(16)


def _make_scatter(d):
  def body(v_hbm, dst2p_hbm, out_hbm, idx_v, zbuf, *bufs):
    rows = bufs[:NBUF_S]
    lsem = bufs[NBUF_S:2 * NBUF_S]
    ssem = bufs[2 * NBUF_S:3 * NBUF_S]
    s_sh = bufs[3 * NBUF_S]
    cid = lax.axis_index("c")
    sid = lax.axis_index("s")
    wid = sid * NC + cid

    for r in range(ZR):
      for col in range(d // 16):
        zbuf[r, pl.ds(col * 16, 16)] = jnp.zeros((16,), jnp.float32)
    for i in range(RPT // ZR):
      pltpu.sync_copy(zbuf, s_sh.at[pl.ds(sid * RPT + i * ZR, ZR)])

    c0 = wid * JPW
    pltpu.sync_copy(dst2p_hbm.at[pl.ds(c0, JPW)], idx_v)
    plsc.subcore_barrier()

    ldesc = [None] * JPW
    sdesc = [None] * JPW

    def load(j):
      s = j % NBUF_S
      off = pl.multiple_of((wid * JPW + j) * CH, 8)
      ldesc[j] = pltpu.async_copy(v_hbm.at[pl.ds(off, CH)], rows[s], lsem[s])

    def scat(j):
      s = j % NBUF_S
      sdesc[j] = pltpu.async_copy(rows[s], s_sh.at[idx_v.at[j]], ssem[s],
                                  add=True)

    for j in range(JPW):
      if j >= NBUF_S:
        sdesc[j - NBUF_S].wait()
      load(j)
      if j >= 1:
        ldesc[j - 1].wait()
        scat(j - 1)
    ldesc[JPW - 1].wait()
    scat(JPW - 1)
    for j in range(JPW - NBUF_S, JPW):
      sdesc[j].wait()

    plsc.subcore_barrier()
    pltpu.sync_copy(
        s_sh.at[pl.ds(sid * RPT, RPT)],
        out_hbm.at[pl.ds(cid * NPAD + sid * RPT, RPT)],
    )

  return pl.kernel(
      body,
      out_type=jax.ShapeDtypeStruct((NC * NPAD, d), jnp.float32),
      mesh=_MESH,
      compiler_params=pltpu.CompilerParams(use_tc_tiling_on_sc=(d == F)),
      scratch_types=(
          [pltpu.VMEM((JPW, CH), jnp.int32),
           pltpu.VMEM((ZR, d), jnp.float32)]
          + [pltpu.VMEM((CH, d), jnp.float32)] * NBUF_S
          + [pltpu.SemaphoreType.DMA] * (2 * NBUF_S)
          + [pltpu.MemorySpace.VMEM_SHARED((NPAD, d), jnp.float32)]
      ),
  )


_scatter_f = _make_scatter(F)
_scatter_g = _make_scatter(16)


_BE = 2000


def _geo_body(ts_ref, td_ref, crel_ref, cm_ref, dist_ref, em_ref, em16_ref):
  ts = ts_ref[...]
  td = td_ref[...]
  diff = ts - td
  d2 = jnp.sum(diff * diff * crel_ref[...], axis=1, keepdims=True)
  dist_ref[...] = jnp.sqrt(d2 + 1e-8)
  am = jnp.sum(ts * cm_ref[...], axis=1, keepdims=True)
  bm = jnp.sum(td * cm_ref[...], axis=1, keepdims=True)
  em = am * bm
  em_ref[...] = em
  em16_ref[...] = jnp.broadcast_to(em, em16_ref.shape)


_geo_call = pl.pallas_call(
    _geo_body,
    grid=(E // _BE,),
    in_specs=[
        pl.BlockSpec((_BE, 16), lambda i: (i, 0)),
        pl.BlockSpec((_BE, 16), lambda i: (i, 0)),
        pl.BlockSpec((1, 16), lambda i: (0, 0)),
        pl.BlockSpec((1, 16), lambda i: (0, 0)),
    ],
    out_specs=[
        pl.BlockSpec((_BE, 1), lambda i: (i, 0)),
        pl.BlockSpec((_BE, 1), lambda i: (i, 0)),
        pl.BlockSpec((_BE, 16), lambda i: (i, 0)),
    ],
    out_shape=[
        jax.ShapeDtypeStruct((E, 1), jnp.float32),
        jax.ShapeDtypeStruct((E, 1), jnp.float32),
        jax.ShapeDtypeStruct((EP, 16), jnp.float32),
    ],
)


_BN = 2000


def _pq_body(node_ref, lat_ref, r_ref, a1a, a1b, a2a, a2b, ag3, b1, p_ref, q_ref):
  node = node_ref[...]
  latv = lat_ref[...]
  rg = jnp.dot(r_ref[...], ag3[...], preferred_element_type=jnp.float32)
  p = (jnp.dot(node, a1a[...], preferred_element_type=jnp.float32)
       + jnp.dot(latv, a1b[...], preferred_element_type=jnp.float32))
  q = (jnp.dot(node, a2a[...], preferred_element_type=jnp.float32)
       + jnp.dot(latv, a2b[...], preferred_element_type=jnp.float32))
  p_ref[...] = p + rg + b1[...]
  q_ref[...] = q - rg


_pq_call = pl.pallas_call(
    _pq_body,
    grid=(N // _BN,),
    in_specs=[
        pl.BlockSpec((_BN, F), lambda i: (i, 0)),
        pl.BlockSpec((_BN, F), lambda i: (i, 0)),
        pl.BlockSpec((_BN, 3), lambda i: (i, 0)),
        pl.BlockSpec((F, F), lambda i: (0, 0)),
        pl.BlockSpec((F, F), lambda i: (0, 0)),
        pl.BlockSpec((F, F), lambda i: (0, 0)),
        pl.BlockSpec((F, F), lambda i: (0, 0)),
        pl.BlockSpec((3, F), lambda i: (0, 0)),
        pl.BlockSpec((1, F), lambda i: (0, 0)),
    ],
    out_specs=[
        pl.BlockSpec((_BN, F), lambda i: (i, 0)),
        pl.BlockSpec((_BN, F), lambda i: (i, 0)),
    ],
    out_shape=[jax.ShapeDtypeStruct((N, F), jnp.float32)] * 2,
)


def _edge_body(gs_ref, gd_ref, z_ref, dist_ref, em_ref, az, we, ag4, be,
               zo_ref, hm_ref):
  z = z_ref[...]
  pre = (gs_ref[...] + gd_ref[...]
         + jnp.dot(z, az[...], preferred_element_type=jnp.float32)
         + dist_ref[...] * ag4[...])
  hmid = jnp.maximum(pre, 0.0)
  em = em_ref[...]
  hm_ref[...] = hmid * em
  zo_ref[...] = z + (jnp.dot(hmid, we[...], preferred_element_type=jnp.float32)
                     + be[...]) * em


def _edge_body_last(gs_ref, gd_ref, z_ref, dist_ref, em_ref, az, ag4, hm_ref):
  pre = (gs_ref[...] + gd_ref[...]
         + jnp.dot(z_ref[...], az[...], preferred_element_type=jnp.float32)
         + dist_ref[...] * ag4[...])
  hm_ref[...] = jnp.maximum(pre, 0.0) * em_ref[...]


_EDGE_IN_SPECS = [
    pl.BlockSpec((_BE, F), lambda i: (i, 0)),
    pl.BlockSpec((_BE, F), lambda i: (i, 0)),
    pl.BlockSpec((_BE, F), lambda i: (i, 0)),
    pl.BlockSpec((_BE, 1), lambda i: (i, 0)),
    pl.BlockSpec((_BE, 1), lambda i: (i, 0)),
    pl.BlockSpec((F, F), lambda i: (0, 0)),
]

_edge_call = pl.pallas_call(
    _edge_body,
    grid=(E // _BE,),
    in_specs=_EDGE_IN_SPECS + [
        pl.BlockSpec((F, F), lambda i: (0, 0)),
        pl.BlockSpec((1, F), lambda i: (0, 0)),
        pl.BlockSpec((1, F), lambda i: (0, 0)),
    ],
    out_specs=[
        pl.BlockSpec((_BE, F), lambda i: (i, 0)),
        pl.BlockSpec((_BE, F), lambda i: (i, 0)),
    ],
    out_shape=[
        jax.ShapeDtypeStruct((E, F), jnp.float32),
        jax.ShapeDtypeStruct((EP, F), jnp.float32),
    ],
)

_edge_call_last = pl.pallas_call(
    _edge_body_last,
    grid=(E // _BE,),
    in_specs=_EDGE_IN_SPECS + [pl.BlockSpec((1, F), lambda i: (0, 0))],
    out_specs=pl.BlockSpec((_BE, F), lambda i: (i, 0)),
    out_shape=jax.ShapeDtypeStruct((EP, F), jnp.float32),
)


def _joint_parts(node, latv, s0, s1, d0, d1, mask, w2, b2k):
  s = s0 + s1
  deg = jnp.sum(d0 + d1, axis=1, keepdims=True) * (1.0 / 16.0)
  agg = (jnp.dot(s, w2, preferred_element_type=jnp.float32) * (1.0 / K)
         + deg * b2k)
  jn = (node + agg[:, 0:F]) * mask
  jl = (latv + agg[:, F:2 * F]) * mask
  jz = agg[:, 2 * F:3 * F] * mask
  return jnp.concatenate([jn, jl, jz], axis=1)


def _node_body(node_ref, lat_ref, s0_ref, s1_ref, d0_ref, d1_ref, mask_ref,
               w2, b2k, wlat, wnode, gamma, beta, lo_ref, no_ref):
  node = node_ref[...]
  latv = lat_ref[...]
  joint = _joint_parts(node, latv, s0_ref[...], s1_ref[...], d0_ref[...],
                       d1_ref[...], mask_ref[...], w2[...], b2k[...])
  lo_ref[...] = latv + jnp.dot(joint, wlat[...],
                               preferred_element_type=jnp.float32)
  npre = node + jnp.dot(joint, wnode[...], preferred_element_type=jnp.float32)
  mu = jnp.mean(npre, axis=1, keepdims=True)
  var = jnp.mean((npre - mu) ** 2, axis=1, keepdims=True)
  no_ref[...] = (npre - mu) / jnp.sqrt(var + 1e-5) * gamma[...] + beta[...]


def _node_body_last(node_ref, lat_ref, s0_ref, s1_ref, d0_ref, d1_ref,
                    mask_ref, w2, b2k, wlat, lo_ref):
  latv = lat_ref[...]
  joint = _joint_parts(node_ref[...], latv, s0_ref[...], s1_ref[...],
                       d0_ref[...], d1_ref[...], mask_ref[...], w2[...],
                       b2k[...])
  lo_ref[...] = latv + jnp.dot(joint, wlat[...],
                               preferred_element_type=jnp.float32)


_NODE_IN_SPECS = [
    pl.BlockSpec((_BN, F), lambda i: (i, 0)),
    pl.BlockSpec((_BN, F), lambda i: (i, 0)),
    pl.BlockSpec((_BN, F), lambda i: (i, 0)),
    pl.BlockSpec((_BN, F), lambda i: (i, 0)),
    pl.BlockSpec((_BN, 16), lambda i: (i, 0)),
    pl.BlockSpec((_BN, 16), lambda i: (i, 0)),
    pl.BlockSpec((_BN, 1), lambda i: (i, 0)),
    pl.BlockSpec((F, 3 * F), lambda i: (0, 0)),
    pl.BlockSpec((1, 3 * F), lambda i: (0, 0)),
    pl.BlockSpec((3 * F, F), lambda i: (0, 0)),
]

_node_call = pl.pallas_call(
    _node_body,
    grid=(N // _BN,),
    in_specs=_NODE_IN_SPECS + [
        pl.BlockSpec((3 * F, F), lambda i: (0, 0)),
        pl.BlockSpec((1, F), lambda i: (0, 0)),
        pl.BlockSpec((1, F), lambda i: (0, 0)),
    ],
    out_specs=[
        pl.BlockSpec((_BN, F), lambda i: (i, 0)),
        pl.BlockSpec((_BN, F), lambda i: (i, 0)),
    ],
    out_shape=[jax.ShapeDtypeStruct((N, F), jnp.float32)] * 2,
)

_node_call_last = pl.pallas_call(
    _node_body_last,
    grid=(N // _BN,),
    in_specs=_NODE_IN_SPECS,
    out_specs=pl.BlockSpec((_BN, F), lambda i: (i, 0)),
    out_shape=jax.ShapeDtypeStruct((N, F), jnp.float32),
)


def kernel(latent_features, node_features, edge_features, rigids_t, node_mask,
           params, edge_index):
  src = edge_index[1]
  dst = edge_index[0]
  padidx = jnp.arange(EP - E, dtype=jnp.int32)
  srcp = jnp.concatenate([src, padidx % N])
  dstp = jnp.concatenate([dst, padidx % N])
  dst2p = jnp.concatenate(
      [dst, N + padidx % (NPAD - N)]).reshape(NCHUNKP, CH)

  tm = jnp.concatenate(
      [rigids_t, node_mask[:, None], jnp.zeros((N, 12), jnp.float32)], axis=1)
  ts, td = _gather_g(tm, tm, srcp, dstp)
  crel = jnp.concatenate(
      [jnp.ones((1, 3), jnp.float32), jnp.zeros((1, 13), jnp.float32)], axis=1)
  cm = jnp.concatenate(
      [jnp.zeros((1, 3), jnp.float32), jnp.ones((1, 1), jnp.float32),
       jnp.zeros((1, 12), jnp.float32)], axis=1)
  dist, em, em16 = _geo_call(ts, td, crel, cm)
  deg16 = _scatter_g(em16, dst2p)

  node = node_features
  lat = latent_features
  z = edge_features
  maskc = node_mask[:, None]

  for l in range(NUM_LAYERS):
    W1 = params['W1'][l]
    a1a, a1b = W1[0:128], W1[128:256]
    a2a, a2b = W1[384:512], W1[512:640]
    az, ag3, ag4 = W1[768:896], W1[896:899], W1[899:900]
    b1 = params['b1'][l][None]
    p, q = _pq_call(node, lat, rigids_t, a1a, a1b, a2a, a2b, ag3, b1)
    gs, gd = _gather_f(p, q, srcp, dstp)
    last = l == NUM_LAYERS - 1
    if last:
      hm = _edge_call_last(gs, gd, z, dist, em, az, ag4)
    else:
      z, hm = _edge_call(gs, gd, z, dist, em, az, params['We'][l],
                         ag4, params['be'][l][None])
    s = _scatter_f(hm, dst2p)
    b2k = (params['b2'][l] / K)[None]
    node_args = (node, lat, s[0:N], s[NPAD:NPAD + N],
                 deg16[0:N], deg16[NPAD:NPAD + N], maskc,
                 params['W2'][l], b2k, params['Wlat'][l])
    if last:
      lat = _node_call_last(*node_args)
    else:
      lat, node = _node_call(*node_args, params['Wnode'][l],
                             params['gamma'][l][None], params['beta'][l][None])
  return lat
